# baseline (speedup 1.0000x reference)
import functools

import jax
import jax.numpy as jnp
from jax import lax
from jax.experimental import pallas as pl
from jax.experimental.pallas import tpu as pltpu
from jax.experimental.pallas import tpu_sc as plsc

NUM_HEADS = 16
OUT = 8


@functools.lru_cache(maxsize=None)
def _make_route(n_total: int, table_n: int):
    info = plsc.get_sparse_core_info()
    nc, ns, lanes = info.num_cores, info.num_subcores, info.num_lanes
    nw = nc * ns
    assert n_total % (nw * lanes) == 0
    c = n_total // nw
    mesh = plsc.VectorSubcoreMesh(core_axis_name="c", subcore_axis_name="s")

    @functools.partial(
        pl.kernel,
        mesh=mesh,
        out_type=jax.ShapeDtypeStruct((n_total,), jnp.int32),
        scratch_types=[
            pltpu.VMEM((c,), jnp.int32),
            pltpu.VMEM((table_n,), jnp.int32),
            pltpu.VMEM((c,), jnp.int32),
        ],
        compiler_params=pltpu.CompilerParams(needs_layout_passes=False),
    )
    def route(bt_hbm, b2h_hbm, out_hbm, bt_v, b2h_v, out_v):
        wid = lax.axis_index("s") * nc + lax.axis_index("c")
        base = wid * c
        pltpu.sync_copy(bt_hbm.at[pl.ds(base, c)], bt_v)
        pltpu.sync_copy(b2h_hbm, b2h_v)
        for i in range(c // lanes):
            idx = bt_v[pl.ds(i * lanes, lanes)]
            out_v[pl.ds(i * lanes, lanes)] = plsc.load_gather(b2h_v, [idx])
        pltpu.sync_copy(out_v, out_hbm.at[pl.ds(base, c)])

    return route


def _decode_block(x_ref, h_ref, w_ref, b_ref, o_ref):
    cols = NUM_HEADS * OUT
    logits = jnp.dot(x_ref[...], w_ref[...], preferred_element_type=jnp.float32)
    logits = logits + b_ref[...]
    hcol = lax.broadcasted_iota(jnp.int32, logits.shape, 1) // OUT
    masked = jnp.where(hcol == h_ref[...], logits, 0.0)
    sel = (lax.broadcasted_iota(jnp.int32, (cols, OUT), 0) % OUT
           == lax.broadcasted_iota(jnp.int32, (cols, OUT), 1)).astype(jnp.float32)
    o_ref[...] = jnp.dot(masked, sel, preferred_element_type=jnp.float32)


def kernel(block_type_grid, x, block2head, W_heads, b_heads):
    b, w, h, l = block_type_grid.shape
    n = b * w * h * l
    d = x.shape[-1]
    cols = NUM_HEADS * OUT

    bf = block_type_grid.reshape(n)
    head_idx = _make_route(n, block2head.shape[0])(bf, block2head)

    x2 = x.reshape(n, d)
    h2 = head_idx.reshape(n, 1)
    w_all = W_heads.transpose(1, 0, 2).reshape(d, cols)
    b_all = b_heads.reshape(1, cols)

    bn = 2048
    out = pl.pallas_call(
        _decode_block,
        grid=(n // bn,),
        in_specs=[
            pl.BlockSpec((bn, d), lambda i: (i, 0)),
            pl.BlockSpec((bn, 1), lambda i: (i, 0)),
            pl.BlockSpec((d, cols), lambda i: (0, 0)),
            pl.BlockSpec((1, cols), lambda i: (0, 0)),
        ],
        out_specs=pl.BlockSpec((bn, OUT), lambda i: (i, 0)),
        out_shape=jax.ShapeDtypeStruct((n, OUT), jnp.float32),
    )(x2, h2, w_all, b_all)
    return out.reshape(b, w, h, l, OUT)


# speedup vs baseline: 1.0717x; 1.0717x over previous
import functools

import jax
import jax.numpy as jnp
from jax import lax
from jax.experimental import pallas as pl
from jax.experimental.pallas import tpu as pltpu
from jax.experimental.pallas import tpu_sc as plsc

NUM_HEADS = 16
OUT = 8


@functools.lru_cache(maxsize=None)
def _make_route(n_total: int, table_n: int):
    info = plsc.get_sparse_core_info()
    nc, ns, lanes = info.num_cores, info.num_subcores, info.num_lanes
    nw = nc * ns
    assert n_total % (nw * lanes) == 0
    c = n_total // nw
    mesh = plsc.VectorSubcoreMesh(core_axis_name="c", subcore_axis_name="s")

    @functools.partial(
        pl.kernel,
        mesh=mesh,
        out_type=jax.ShapeDtypeStruct((n_total,), jnp.int32),
        scratch_types=[
            pltpu.VMEM((c,), jnp.int32),
            pltpu.VMEM((table_n,), jnp.int32),
            pltpu.VMEM((c,), jnp.int32),
        ],
        compiler_params=pltpu.CompilerParams(needs_layout_passes=False),
    )
    def route(bt_hbm, b2h_hbm, out_hbm, bt_v, b2h_v, out_v):
        wid = lax.axis_index("s") * nc + lax.axis_index("c")
        base = wid * c
        pltpu.sync_copy(bt_hbm.at[pl.ds(base, c)], bt_v)
        pltpu.sync_copy(b2h_hbm, b2h_v)
        for i in range(c // lanes):
            idx = bt_v[pl.ds(i * lanes, lanes)]
            out_v[pl.ds(i * lanes, lanes)] = plsc.load_gather(b2h_v, [idx])
        pltpu.sync_copy(out_v, out_hbm.at[pl.ds(base, c)])

    return route


def _decode_block(x_ref, h_ref, w_ref, b_ref, o_ref):
    cols = NUM_HEADS * OUT
    bn = h_ref.shape[0]
    x = x_ref[...].reshape(bn, x_ref.shape[-1])
    logits = jnp.dot(x, w_ref[...], preferred_element_type=jnp.float32)
    logits = logits + b_ref[...]
    hcol = lax.broadcasted_iota(jnp.int32, logits.shape, 1) // OUT
    masked = jnp.where(hcol == h_ref[...].astype(jnp.int32), logits, 0.0)
    sel = (lax.broadcasted_iota(jnp.int32, (cols, OUT), 0) % OUT
           == lax.broadcasted_iota(jnp.int32, (cols, OUT), 1)).astype(jnp.float32)
    res = jnp.dot(masked, sel, preferred_element_type=jnp.float32)
    o_ref[...] = res.reshape(o_ref.shape)


def kernel(block_type_grid, x, block2head, W_heads, b_heads):
    b, w, h, l = block_type_grid.shape
    n = b * w * h * l
    d = x.shape[-1]
    cols = NUM_HEADS * OUT
    bn = n // b

    bf = block_type_grid.reshape(n)
    head_idx = _make_route(n, block2head.shape[0])(bf, block2head)

    h2 = head_idx.astype(jnp.int8).reshape(n, 1)
    w_all = W_heads.transpose(1, 0, 2).reshape(d, cols)
    b_all = b_heads.reshape(1, cols)

    out = pl.pallas_call(
        _decode_block,
        grid=(b,),
        in_specs=[
            pl.BlockSpec((1, w, h, l, d), lambda i: (i, 0, 0, 0, 0)),
            pl.BlockSpec((bn, 1), lambda i: (i, 0)),
            pl.BlockSpec((d, cols), lambda i: (0, 0)),
            pl.BlockSpec((1, cols), lambda i: (0, 0)),
        ],
        out_specs=pl.BlockSpec((1, w, h, l, OUT), lambda i: (i, 0, 0, 0, 0)),
        out_shape=jax.ShapeDtypeStruct((b, w, h, l, OUT), jnp.float32),
    )(x, h2, w_all, b_all)
    return out


# speedup vs baseline: 1.5329x; 1.4304x over previous
import functools

import jax
import jax.numpy as jnp
from jax import lax
from jax.experimental import pallas as pl
from jax.experimental.pallas import tpu as pltpu
from jax.experimental.pallas import tpu_sc as plsc

NUM_HEADS = 16
OUT = 8


@functools.lru_cache(maxsize=None)
def _make_route(n_total: int, table_n: int):
    info = plsc.get_sparse_core_info()
    nc, ns, lanes = info.num_cores, info.num_subcores, info.num_lanes
    nw = nc * ns
    assert n_total % (nw * lanes) == 0
    c = n_total // nw
    mesh = plsc.VectorSubcoreMesh(core_axis_name="c", subcore_axis_name="s")

    @functools.partial(
        pl.kernel,
        mesh=mesh,
        out_type=jax.ShapeDtypeStruct((n_total,), jnp.int32),
        scratch_types=[
            pltpu.VMEM((c,), jnp.int32),
            pltpu.VMEM((table_n,), jnp.int32),
            pltpu.VMEM((c,), jnp.int32),
        ],
        compiler_params=pltpu.CompilerParams(needs_layout_passes=False),
    )
    def route(bt_hbm, b2h_hbm, out_hbm, bt_v, b2h_v, out_v):
        wid = lax.axis_index("s") * nc + lax.axis_index("c")
        base = wid * c
        pltpu.sync_copy(bt_hbm.at[pl.ds(base, c)], bt_v)
        pltpu.sync_copy(b2h_hbm, b2h_v)
        for i in range(c // lanes):
            idx = bt_v[pl.ds(i * lanes, lanes)]
            out_v[pl.ds(i * lanes, lanes)] = plsc.load_gather(b2h_v, [idx])
        pltpu.sync_copy(out_v, out_hbm.at[pl.ds(base, c)])

    return route


def _decode_block(x_ref, h_ref, w_ref, b_ref, o_ref):
    cols = NUM_HEADS * OUT
    bn = h_ref.shape[0]
    x = x_ref[...].reshape(bn, x_ref.shape[-1])
    logits = jnp.dot(x, w_ref[...], preferred_element_type=jnp.float32)
    logits = logits + b_ref[...]
    hcol = lax.broadcasted_iota(jnp.int32, logits.shape, 1) // OUT
    masked = jnp.where(hcol == h_ref[...].reshape(bn, 1), logits, 0.0)
    sel = (lax.broadcasted_iota(jnp.int32, (cols, OUT), 0) % OUT
           == lax.broadcasted_iota(jnp.int32, (cols, OUT), 1)).astype(jnp.float32)
    res = jnp.dot(masked, sel, preferred_element_type=jnp.float32)
    res_t = jnp.swapaxes(res.reshape(bn // 16, 16, OUT), 1, 2)
    o_ref[...] = res_t.reshape(o_ref.shape)


def kernel(block_type_grid, x, block2head, W_heads, b_heads):
    b, w, h, l = block_type_grid.shape
    n = b * w * h * l
    d = x.shape[-1]
    cols = NUM_HEADS * OUT
    bn = n // b

    bf = block_type_grid.reshape(n)
    head_idx = _make_route(n, block2head.shape[0])(bf, block2head)

    w_all = W_heads.transpose(1, 0, 2).reshape(d, cols)
    b_all = b_heads.reshape(1, cols)

    out = pl.pallas_call(
        _decode_block,
        grid=(b,),
        in_specs=[
            pl.BlockSpec((1, w, h, l, d), lambda i: (i, 0, 0, 0, 0)),
            pl.BlockSpec((bn,), lambda i: (i,)),
            pl.BlockSpec((d, cols), lambda i: (0, 0)),
            pl.BlockSpec((1, cols), lambda i: (0, 0)),
        ],
        out_specs=pl.BlockSpec((1, w, h, OUT, l), lambda i: (i, 0, 0, 0, 0)),
        out_shape=jax.ShapeDtypeStruct((b, w, h, OUT, l), jnp.float32),
    )(x, head_idx, w_all, b_all)
    return jnp.swapaxes(out, 3, 4)
